# baseline (speedup 1.0000x reference)
import jax
import jax.numpy as jnp
from jax.experimental import pallas as pl


_COLS = 128
_BLOCK_ROWS = 3128


def _body(p_ref, a_ref, b_ref, o_ref):
    o_ref[...] = a_ref[...] + b_ref[...] * jnp.maximum(p_ref[...], 0.0)


def kernel(products, A, B):
    n = products.shape[0]
    rows = n // _COLS
    p2 = products.reshape(rows, _COLS)
    a2 = A.reshape(rows, _COLS)
    b2 = B.reshape(rows, _COLS)
    grid = (rows + _BLOCK_ROWS - 1) // _BLOCK_ROWS
    spec = pl.BlockSpec((_BLOCK_ROWS, _COLS), lambda i: (i, 0))
    out = pl.pallas_call(
        _body,
        grid=(grid,),
        in_specs=[spec, spec, spec],
        out_specs=spec,
        out_shape=jax.ShapeDtypeStruct((rows, _COLS), jnp.float32),
    )(p2, a2, b2)
    return out.reshape(n)


# speedup vs baseline: 1.0005x; 1.0005x over previous
import jax
import jax.numpy as jnp
from jax.experimental import pallas as pl


_COLS = 128
_BLOCK_ROWS = 4168


def _body(p_ref, a_ref, b_ref, o_ref):
    o_ref[...] = a_ref[...] + b_ref[...] * jnp.maximum(p_ref[...], 0.0)


def kernel(products, A, B):
    n = products.shape[0]
    rows = n // _COLS
    p2 = products.reshape(rows, _COLS)
    a2 = A.reshape(rows, _COLS)
    b2 = B.reshape(rows, _COLS)
    grid = (rows + _BLOCK_ROWS - 1) // _BLOCK_ROWS
    spec = pl.BlockSpec((_BLOCK_ROWS, _COLS), lambda i: (i, 0))
    out = pl.pallas_call(
        _body,
        grid=(grid,),
        in_specs=[spec, spec, spec],
        out_specs=spec,
        out_shape=jax.ShapeDtypeStruct((rows, _COLS), jnp.float32),
    )(p2, a2, b2)
    return out.reshape(n)
